# baseline (speedup 1.0000x reference)
import functools

import jax
import jax.numpy as jnp
from jax import lax
from jax.experimental import pallas as pl
from jax.experimental.pallas import tpu as pltpu
from jax.experimental.pallas import tpu_sc as plsc

_N = 10000
_E = 320000
_D = 128
_NP = 10240
_NC = 2
_NS = 16
_NW = _NC * _NS
_EW = _E // _NW
_C = 80
_NCHUNK = _EW // _C
_RT = _NP // _NS
_FB = 25

@functools.lru_cache(maxsize=None)
def _sc_mesh():
    return plsc.VectorSubcoreMesh(core_axis_name="c", subcore_axis_name="s",
                                  num_cores=_NC, num_subcores=_NS)


def _sc_deg_body(col4_hbm, zeros_hbm, deg_out, colidx, onesbuf, sem, degsh):
    cid = lax.axis_index("c")
    sid = lax.axis_index("s")
    wid = cid * _NS + sid
    for j in range(_C // 16):
        onesbuf[pl.ds(j * 16, 16)] = jnp.ones((16,), jnp.float32)
    pltpu.sync_copy(zeros_hbm.at[pl.ds(sid * _RT, _RT)],
                    degsh.at[pl.ds(sid * _RT, _RT)])
    plsc.subcore_barrier()

    @pl.loop(0, _NCHUNK // _FB)
    def _chunk(i):
        pltpu.sync_copy(col4_hbm.at[wid, i], colidx)
        for j in range(_FB):
            pltpu.async_copy(onesbuf, degsh.at[colidx.at[j]], sem, add=True)
        for j in range(_FB):
            pltpu.make_async_copy(onesbuf, degsh.at[colidx.at[j]],
                                  sem).wait()

    plsc.subcore_barrier()
    pltpu.sync_copy(degsh.at[pl.ds(sid * _RT, _RT)],
                    deg_out.at[pl.ds(cid * _NP + sid * _RT, _RT)])


@functools.lru_cache(maxsize=None)
def _sc_deg():
    return pl.kernel(
        _sc_deg_body,
        out_type=jax.ShapeDtypeStruct((_NC * _NP,), jnp.float32),
        mesh=_sc_mesh(),
        scratch_types=[
            pltpu.VMEM((_FB, _C), jnp.int32),
            pltpu.VMEM((_C,), jnp.float32),
            pltpu.SemaphoreType.DMA,
            pltpu.VMEM_SHARED((_NP,), jnp.float32),
        ],
    )


def _sc_agg_body(hs_hbm, row2_hbm, col3_hbm, zeros_hbm, out_hbm,
                 rowflat, colidx, rows01, g0, g1, aggsh):
    cid = lax.axis_index("c")
    sid = lax.axis_index("s")
    wid = cid * _NS + sid
    rows0 = rows01.at[pl.ds(0, _C)]
    rows1 = rows01.at[pl.ds(_C, _C)]
    pltpu.sync_copy(zeros_hbm.at[pl.ds(sid * _RT, _RT)],
                    aggsh.at[pl.ds(sid * _RT, _RT)])
    pltpu.sync_copy(row2_hbm.at[wid], rowflat)
    pltpu.sync_copy(col3_hbm.at[wid], colidx)
    plsc.subcore_barrier()

    pltpu.async_copy(hs_hbm.at[rowflat.at[pl.ds(0, _C)]], rows0, g0)
    pltpu.async_copy(hs_hbm.at[rowflat.at[pl.ds(_C, _C)]], rows1, g1)

    @pl.loop(0, _NCHUNK - 1, step=2)
    def _chunk(i):
        pltpu.make_async_copy(
            hs_hbm.at[rowflat.at[pl.ds(i * _C, _C)]], rows0, g0).wait()
        pltpu.sync_copy(rows0, aggsh.at[colidx.at[i]], add=True)
        pltpu.async_copy(
            hs_hbm.at[rowflat.at[pl.ds((i + 2) * _C, _C)]], rows0, g0)
        pltpu.make_async_copy(
            hs_hbm.at[rowflat.at[pl.ds((i + 1) * _C, _C)]], rows1, g1).wait()
        pltpu.sync_copy(rows1, aggsh.at[colidx.at[i + 1]], add=True)

        @pl.when(i + 3 < _NCHUNK)
        def _():
            pltpu.async_copy(
                hs_hbm.at[rowflat.at[pl.ds((i + 3) * _C, _C)]], rows1, g1)

    pltpu.make_async_copy(
        hs_hbm.at[rowflat.at[pl.ds((_NCHUNK - 1) * _C, _C)]], rows0, g0).wait()
    pltpu.sync_copy(rows0, aggsh.at[colidx.at[_NCHUNK - 1]], add=True)

    plsc.subcore_barrier()
    pltpu.sync_copy(aggsh.at[pl.ds(sid * _RT, _RT)],
                    out_hbm.at[pl.ds(cid * _NP + sid * _RT, _RT)])


@functools.lru_cache(maxsize=None)
def _sc_agg():
    return pl.kernel(
        _sc_agg_body,
        out_type=jax.ShapeDtypeStruct((_NC * _NP, _D), jnp.float32),
        mesh=_sc_mesh(),
        scratch_types=[
            pltpu.VMEM((_EW,), jnp.int32),
            pltpu.VMEM((_NCHUNK, _C), jnp.int32),
            pltpu.VMEM((2 * _C, _D), jnp.float32),
            pltpu.SemaphoreType.DMA,
            pltpu.SemaphoreType.DMA,
            pltpu.VMEM_SHARED((_NP, _D), jnp.float32),
        ],
    )


_BN = 1280


def _tc_prep_body(d0_ref, d1_ref, h_ref, w_ref, b_ref, dis_ref, hs_ref,
                  xb_ref):
    deg = d0_ref[...] + d1_ref[...]
    dis = jnp.where(deg > 0, lax.rsqrt(deg), 0.0)
    dis_ref[...] = dis
    hs_ref[...] = dis * h_ref[...]
    xb_ref[...] = jnp.dot(h_ref[...], w_ref[...],
                          preferred_element_type=jnp.float32) + b_ref[...]


def _tc_prep(d0, d1, h, w2, b):
    grid = _NP // _BN
    return pl.pallas_call(
        _tc_prep_body,
        grid=(grid,),
        in_specs=[
            pl.BlockSpec((_BN, 1), lambda i: (i, 0)),
            pl.BlockSpec((_BN, 1), lambda i: (i, 0)),
            pl.BlockSpec((_BN, _D), lambda i: (i, 0)),
            pl.BlockSpec((_D, _D), lambda i: (0, 0)),
            pl.BlockSpec((1, _D), lambda i: (0, 0)),
        ],
        out_specs=[
            pl.BlockSpec((_BN, 1), lambda i: (i, 0)),
            pl.BlockSpec((_BN, _D), lambda i: (i, 0)),
            pl.BlockSpec((_BN, _D), lambda i: (i, 0)),
        ],
        out_shape=[
            jax.ShapeDtypeStruct((_NP, 1), jnp.float32),
            jax.ShapeDtypeStruct((_NP, _D), jnp.float32),
            jax.ShapeDtypeStruct((_NP, _D), jnp.float32),
        ],
    )(d0, d1, h, w2, b)


def _tc_layer_body(p0_ref, p1_ref, dis_ref, xb_ref, w_ref, wn_ref, bn_ref,
                   h_out_ref, hs_out_ref, xbn_ref):
    raw = p0_ref[...] + p1_ref[...]
    dis = dis_ref[...]
    agg = dis * raw
    acc = jnp.dot(agg, w_ref[...], preferred_element_type=jnp.float32)
    acc = jnp.maximum(acc + xb_ref[...], 0.0)
    h_out_ref[...] = acc
    hs_out_ref[...] = dis * acc
    xbn_ref[...] = jnp.dot(acc, wn_ref[...],
                           preferred_element_type=jnp.float32) + bn_ref[...]


def _tc_layer(p0, p1, dis, xb, w1, wn2, bn):
    grid = _NP // _BN
    return pl.pallas_call(
        _tc_layer_body,
        grid=(grid,),
        in_specs=[
            pl.BlockSpec((_BN, _D), lambda i: (i, 0)),
            pl.BlockSpec((_BN, _D), lambda i: (i, 0)),
            pl.BlockSpec((_BN, 1), lambda i: (i, 0)),
            pl.BlockSpec((_BN, _D), lambda i: (i, 0)),
            pl.BlockSpec((_D, _D), lambda i: (0, 0)),
            pl.BlockSpec((_D, _D), lambda i: (0, 0)),
            pl.BlockSpec((1, _D), lambda i: (0, 0)),
        ],
        out_specs=[
            pl.BlockSpec((_BN, _D), lambda i: (i, 0)),
            pl.BlockSpec((_BN, _D), lambda i: (i, 0)),
            pl.BlockSpec((_BN, _D), lambda i: (i, 0)),
        ],
        out_shape=[
            jax.ShapeDtypeStruct((_NP, _D), jnp.float32),
            jax.ShapeDtypeStruct((_NP, _D), jnp.float32),
            jax.ShapeDtypeStruct((_NP, _D), jnp.float32),
        ],
    )(p0, p1, dis, xb, w1, wn2, bn)


def _tc_final_body(p0_ref, p1_ref, dis_ref, xb_ref, w_ref, h_out_ref):
    raw = p0_ref[...] + p1_ref[...]
    agg = dis_ref[...] * raw
    acc = jnp.dot(agg, w_ref[...], preferred_element_type=jnp.float32)
    h_out_ref[...] = acc + xb_ref[...]


def _tc_final(p0, p1, dis, xb, w1):
    grid = _NP // _BN
    return pl.pallas_call(
        _tc_final_body,
        grid=(grid,),
        in_specs=[
            pl.BlockSpec((_BN, _D), lambda i: (i, 0)),
            pl.BlockSpec((_BN, _D), lambda i: (i, 0)),
            pl.BlockSpec((_BN, 1), lambda i: (i, 0)),
            pl.BlockSpec((_BN, _D), lambda i: (i, 0)),
            pl.BlockSpec((_D, _D), lambda i: (0, 0)),
        ],
        out_specs=pl.BlockSpec((_BN, _D), lambda i: (i, 0)),
        out_shape=jax.ShapeDtypeStruct((_NP, _D), jnp.float32),
    )(p0, p1, dis, xb, w1)


def kernel(h, edge_index, W0, b0, W1, b1, W2, b2):
    row2 = edge_index[0].reshape(_NW, _EW)
    col3 = edge_index[1].reshape(_NW, _NCHUNK, _C)
    hpad = jnp.zeros((_NP, _D), jnp.float32).at[:_N].set(h)
    z1 = jnp.zeros((_NP,), jnp.float32)
    z2 = jnp.zeros((_NP, _D), jnp.float32)

    col4 = edge_index[1].reshape(_NW, _NCHUNK // _FB, _FB, _C)
    deg_parts = _sc_deg()(col4, z1)
    d0 = deg_parts[:_NP].reshape(_NP, 1)
    d1 = deg_parts[_NP:].reshape(_NP, 1)
    dis, hs, xb = _tc_prep(d0, d1, hpad, W0[_D:], b0.reshape(1, _D))

    for w, wn, bn in ((W0, W1, b1), (W1, W2, b2)):
        parts = _sc_agg()(hs, row2, col3, z2)
        _, hs, xb = _tc_layer(parts[:_NP], parts[_NP:], dis, xb,
                              w[:_D], wn[_D:], bn.reshape(1, _D))
    parts = _sc_agg()(hs, row2, col3, z2)
    out = _tc_final(parts[:_NP], parts[_NP:], dis, xb, W2[:_D])
    return out[:_N]


# speedup vs baseline: 1.0217x; 1.0217x over previous
import functools

import jax
import jax.numpy as jnp
from jax import lax
from jax.experimental import pallas as pl
from jax.experimental.pallas import tpu as pltpu
from jax.experimental.pallas import tpu_sc as plsc

_N = 10000
_E = 320000
_D = 128
_NP = 10240
_NC = 2
_NS = 16
_NW = _NC * _NS
_EW = _E // _NW
_C = 80
_NCHUNK = _EW // _C
_RT = _NP // _NS
_FB = 25

@functools.lru_cache(maxsize=None)
def _sc_mesh():
    return plsc.VectorSubcoreMesh(core_axis_name="c", subcore_axis_name="s",
                                  num_cores=_NC, num_subcores=_NS)


def _sc_deg_body(col4_hbm, zeros_hbm, deg_out, colidx, onesbuf, sem, degsh):
    cid = lax.axis_index("c")
    sid = lax.axis_index("s")
    wid = cid * _NS + sid
    for j in range(_C // 16):
        onesbuf[pl.ds(j * 16, 16)] = jnp.ones((16,), jnp.float32)
    pltpu.sync_copy(zeros_hbm.at[pl.ds(sid * _RT, _RT)],
                    degsh.at[pl.ds(sid * _RT, _RT)])
    plsc.subcore_barrier()

    @pl.loop(0, _NCHUNK // _FB)
    def _chunk(i):
        pltpu.sync_copy(col4_hbm.at[wid, i], colidx)
        for j in range(_FB):
            pltpu.async_copy(onesbuf, degsh.at[colidx.at[j]], sem, add=True)
        for j in range(_FB):
            pltpu.make_async_copy(onesbuf, degsh.at[colidx.at[j]],
                                  sem).wait()

    plsc.subcore_barrier()
    pltpu.sync_copy(degsh.at[pl.ds(sid * _RT, _RT)],
                    deg_out.at[pl.ds(cid * _NP + sid * _RT, _RT)])


@functools.lru_cache(maxsize=None)
def _sc_deg():
    return pl.kernel(
        _sc_deg_body,
        out_type=jax.ShapeDtypeStruct((_NC * _NP,), jnp.float32),
        mesh=_sc_mesh(),
        scratch_types=[
            pltpu.VMEM((_FB, _C), jnp.int32),
            pltpu.VMEM((_C,), jnp.float32),
            pltpu.SemaphoreType.DMA,
            pltpu.VMEM_SHARED((_NP,), jnp.float32),
        ],
    )


def _sc_agg_body(hs_hbm, row2_hbm, col3_hbm, zeros_hbm, out_hbm,
                 rowflat, colidx, rows01, g0, g1, zs, aggsh):
    cid = lax.axis_index("c")
    sid = lax.axis_index("s")
    wid = cid * _NS + sid
    rows0 = rows01.at[pl.ds(0, _C)]
    rows1 = rows01.at[pl.ds(_C, _C)]
    pltpu.async_copy(zeros_hbm.at[pl.ds(sid * _RT, _RT)],
                     aggsh.at[pl.ds(sid * _RT, _RT)], zs)
    pltpu.sync_copy(row2_hbm.at[wid], rowflat)
    pltpu.sync_copy(col3_hbm.at[wid], colidx)

    pltpu.async_copy(hs_hbm.at[rowflat.at[pl.ds(0, _C)]], rows0, g0)
    pltpu.async_copy(hs_hbm.at[rowflat.at[pl.ds(_C, _C)]], rows1, g1)
    pltpu.make_async_copy(zeros_hbm.at[pl.ds(sid * _RT, _RT)],
                          aggsh.at[pl.ds(sid * _RT, _RT)], zs).wait()
    plsc.subcore_barrier()

    @pl.loop(0, _NCHUNK - 1, step=2)
    def _chunk(i):
        pltpu.make_async_copy(
            hs_hbm.at[rowflat.at[pl.ds(i * _C, _C)]], rows0, g0).wait()
        pltpu.sync_copy(rows0, aggsh.at[colidx.at[i]], add=True)
        pltpu.async_copy(
            hs_hbm.at[rowflat.at[pl.ds((i + 2) * _C, _C)]], rows0, g0)
        pltpu.make_async_copy(
            hs_hbm.at[rowflat.at[pl.ds((i + 1) * _C, _C)]], rows1, g1).wait()
        pltpu.sync_copy(rows1, aggsh.at[colidx.at[i + 1]], add=True)

        @pl.when(i + 3 < _NCHUNK)
        def _():
            pltpu.async_copy(
                hs_hbm.at[rowflat.at[pl.ds((i + 3) * _C, _C)]], rows1, g1)

    pltpu.make_async_copy(
        hs_hbm.at[rowflat.at[pl.ds((_NCHUNK - 1) * _C, _C)]], rows0, g0).wait()
    pltpu.sync_copy(rows0, aggsh.at[colidx.at[_NCHUNK - 1]], add=True)

    plsc.subcore_barrier()
    pltpu.sync_copy(aggsh.at[pl.ds(sid * _RT, _RT)],
                    out_hbm.at[pl.ds(cid * _NP + sid * _RT, _RT)])


@functools.lru_cache(maxsize=None)
def _sc_agg():
    return pl.kernel(
        _sc_agg_body,
        out_type=jax.ShapeDtypeStruct((_NC * _NP, _D), jnp.float32),
        mesh=_sc_mesh(),
        scratch_types=[
            pltpu.VMEM((_EW,), jnp.int32),
            pltpu.VMEM((_NCHUNK, _C), jnp.int32),
            pltpu.VMEM((2 * _C, _D), jnp.float32),
            pltpu.SemaphoreType.DMA,
            pltpu.SemaphoreType.DMA,
            pltpu.SemaphoreType.DMA,
            pltpu.VMEM_SHARED((_NP, _D), jnp.float32),
        ],
    )


_BN = 1280


def _tc_prep_body(d0_ref, d1_ref, h_ref, w_ref, b_ref, dis_ref, hs_ref,
                  xb_ref):
    deg = d0_ref[...] + d1_ref[...]
    dis = jnp.where(deg > 0, lax.rsqrt(deg), 0.0)
    dis_ref[...] = dis
    hs_ref[...] = dis * h_ref[...]
    xb_ref[...] = jnp.dot(h_ref[...], w_ref[...],
                          preferred_element_type=jnp.float32) + b_ref[...]


def _tc_prep(d0, d1, h, w2, b):
    grid = _NP // _BN
    return pl.pallas_call(
        _tc_prep_body,
        grid=(grid,),
        in_specs=[
            pl.BlockSpec((_BN, 1), lambda i: (i, 0)),
            pl.BlockSpec((_BN, 1), lambda i: (i, 0)),
            pl.BlockSpec((_BN, _D), lambda i: (i, 0)),
            pl.BlockSpec((_D, _D), lambda i: (0, 0)),
            pl.BlockSpec((1, _D), lambda i: (0, 0)),
        ],
        out_specs=[
            pl.BlockSpec((_BN, 1), lambda i: (i, 0)),
            pl.BlockSpec((_BN, _D), lambda i: (i, 0)),
            pl.BlockSpec((_BN, _D), lambda i: (i, 0)),
        ],
        out_shape=[
            jax.ShapeDtypeStruct((_NP, 1), jnp.float32),
            jax.ShapeDtypeStruct((_NP, _D), jnp.float32),
            jax.ShapeDtypeStruct((_NP, _D), jnp.float32),
        ],
    )(d0, d1, h, w2, b)


def _tc_layer_body(p0_ref, p1_ref, dis_ref, xb_ref, w_ref, wn_ref, bn_ref,
                   h_out_ref, hs_out_ref, xbn_ref):
    raw = p0_ref[...] + p1_ref[...]
    dis = dis_ref[...]
    agg = dis * raw
    acc = jnp.dot(agg, w_ref[...], preferred_element_type=jnp.float32)
    acc = jnp.maximum(acc + xb_ref[...], 0.0)
    h_out_ref[...] = acc
    hs_out_ref[...] = dis * acc
    xbn_ref[...] = jnp.dot(acc, wn_ref[...],
                           preferred_element_type=jnp.float32) + bn_ref[...]


def _tc_layer(p0, p1, dis, xb, w1, wn2, bn):
    grid = _NP // _BN
    return pl.pallas_call(
        _tc_layer_body,
        grid=(grid,),
        in_specs=[
            pl.BlockSpec((_BN, _D), lambda i: (i, 0)),
            pl.BlockSpec((_BN, _D), lambda i: (i, 0)),
            pl.BlockSpec((_BN, 1), lambda i: (i, 0)),
            pl.BlockSpec((_BN, _D), lambda i: (i, 0)),
            pl.BlockSpec((_D, _D), lambda i: (0, 0)),
            pl.BlockSpec((_D, _D), lambda i: (0, 0)),
            pl.BlockSpec((1, _D), lambda i: (0, 0)),
        ],
        out_specs=[
            pl.BlockSpec((_BN, _D), lambda i: (i, 0)),
            pl.BlockSpec((_BN, _D), lambda i: (i, 0)),
            pl.BlockSpec((_BN, _D), lambda i: (i, 0)),
        ],
        out_shape=[
            jax.ShapeDtypeStruct((_NP, _D), jnp.float32),
            jax.ShapeDtypeStruct((_NP, _D), jnp.float32),
            jax.ShapeDtypeStruct((_NP, _D), jnp.float32),
        ],
    )(p0, p1, dis, xb, w1, wn2, bn)


def _tc_final_body(p0_ref, p1_ref, dis_ref, xb_ref, w_ref, h_out_ref):
    raw = p0_ref[...] + p1_ref[...]
    agg = dis_ref[...] * raw
    acc = jnp.dot(agg, w_ref[...], preferred_element_type=jnp.float32)
    h_out_ref[...] = acc + xb_ref[...]


def _tc_final(p0, p1, dis, xb, w1):
    grid = _NP // _BN
    return pl.pallas_call(
        _tc_final_body,
        grid=(grid,),
        in_specs=[
            pl.BlockSpec((_BN, _D), lambda i: (i, 0)),
            pl.BlockSpec((_BN, _D), lambda i: (i, 0)),
            pl.BlockSpec((_BN, 1), lambda i: (i, 0)),
            pl.BlockSpec((_BN, _D), lambda i: (i, 0)),
            pl.BlockSpec((_D, _D), lambda i: (0, 0)),
        ],
        out_specs=pl.BlockSpec((_BN, _D), lambda i: (i, 0)),
        out_shape=jax.ShapeDtypeStruct((_NP, _D), jnp.float32),
    )(p0, p1, dis, xb, w1)


def kernel(h, edge_index, W0, b0, W1, b1, W2, b2):
    row2 = edge_index[0].reshape(_NW, _EW)
    col3 = edge_index[1].reshape(_NW, _NCHUNK, _C)
    hpad = jnp.zeros((_NP, _D), jnp.float32).at[:_N].set(h)
    z1 = jnp.zeros((_NP,), jnp.float32)
    z2 = jnp.zeros((_NP, _D), jnp.float32)

    col4 = edge_index[1].reshape(_NW, _NCHUNK // _FB, _FB, _C)
    deg_parts = _sc_deg()(col4, z1)
    d0 = deg_parts[:_NP].reshape(_NP, 1)
    d1 = deg_parts[_NP:].reshape(_NP, 1)
    dis, hs, xb = _tc_prep(d0, d1, hpad, W0[_D:], b0.reshape(1, _D))

    for w, wn, bn in ((W0, W1, b1), (W1, W2, b2)):
        parts = _sc_agg()(hs, row2, col3, z2)
        _, hs, xb = _tc_layer(parts[:_NP], parts[_NP:], dis, xb,
                              w[:_D], wn[_D:], bn.reshape(1, _D))
    parts = _sc_agg()(hs, row2, col3, z2)
    out = _tc_final(parts[:_NP], parts[_NP:], dis, xb, W2[:_D])
    return out[:_N]
